# baseline (speedup 1.0000x reference)
import jax
import jax.numpy as jnp
from jax import lax
from jax.experimental import pallas as pl
from jax.experimental.pallas import tpu as pltpu
from jax.experimental.pallas import tpu_sc as plsc

B_TOTAL = 16384
L = 50
D = 16
C = 3
LANES = 16
NUM_CORES = 2
NUM_SUBCORES = 16
NW = NUM_CORES * NUM_SUBCORES
ROWS_PER_TILE = B_TOTAL // NW
CHUNK_ROWS = 64
GROUPS = CHUNK_ROWS // LANES
CHUNKS_PER_TILE = ROWS_PER_TILE // CHUNK_ROWS
LOOKUPS = CHUNK_ROWS * L
IDX_W = 128
NIDX = LOOKUPS // IDX_W

LN2 = 0.6931471805599453

TCOLS = EMB_ROWS = 1000000
TBLK = 1000
NBLK_TOTAL = TCOLS // TBLK
MAX_BLK_PER_TILE = -(-NBLK_TOTAL // NW)
TUNROLL = 10


def _transpose_body(embt_hbm, out_hbm, in_v, out_v):
    cid = lax.axis_index("c")
    sid = lax.axis_index("s")
    wid = sid * NUM_CORES + cid
    lanes = lax.iota(jnp.int32, LANES)

    def blk_body(b, carry):
        blk = b * NW + wid

        @pl.when(blk < NBLK_TOTAL)
        def _():
            col0 = blk * TBLK
            pltpu.sync_copy(embt_hbm.at[:, pl.ds(col0, TBLK)], in_v)

            def col_body(i, carry):
                c = i * TUNROLL
                for k in range(TUNROLL):
                    vec = plsc.load_gather(
                        in_v, [lanes, jnp.full((LANES,), c + k, jnp.int32)])
                    out_v[pl.ds((c + k) * D, D)] = vec
                return carry

            lax.fori_loop(0, TBLK // TUNROLL, col_body, 0)
            pltpu.sync_copy(out_v, out_hbm.at[pl.ds(col0 * D, TBLK * D)])

        return carry

    lax.fori_loop(0, MAX_BLK_PER_TILE, blk_body, 0)


RBLK = 8
ROW_STRIDE = 32


def _body(x_hbm, emb_hbm, w_hbm, b_hbm, out_hbm,
          idx_v0, idx_v1, rows_v0, rows_v1, w_v, b_v, acc_v, out_v,
          sem0, sem1):
    cid = lax.axis_index("c")
    sid = lax.axis_index("s")
    wid = sid * NUM_CORES + cid

    pltpu.sync_copy(w_hbm, w_v)
    pltpu.sync_copy(b_hbm, b_v)
    lanes = lax.iota(jnp.int32, LANES)
    lanes48 = lanes * (C * D)

    def issue(g, idx_v, rows_v, sem):
        chunk = wid * CHUNKS_PER_TILE + g
        pltpu.sync_copy(x_hbm.at[pl.ds(chunk * NIDX, NIDX)], idx_v)
        for j in range(NIDX):
            pltpu.async_copy(emb_hbm.at[idx_v.at[j]],
                             rows_v.at[pl.ds(j * IDX_W, IDX_W)], sem)

    def drain(idx_v, rows_v, sem):
        for j in range(NIDX):
            pltpu.make_async_copy(emb_hbm.at[idx_v.at[j]],
                                  rows_v.at[pl.ds(j * IDX_W, IDX_W)],
                                  sem).wait()

    def compute(g, rows_v):
        chunk = wid * CHUNKS_PER_TILE + g
        for blk in range(CHUNK_ROWS // RBLK):
            def l_body(l, accs, blk=blk):
                accs = list(accs)
                w0 = w_v[pl.ds((l * C + 0) * D, D)]
                w1 = w_v[pl.ds((l * C + 1) * D, D)]
                w2 = w_v[pl.ds((l * C + 2) * D, D)]
                for j in range(RBLK):
                    r = blk * RBLK + j
                    e = rows_v[r * L + l]
                    accs[j * C + 0] = accs[j * C + 0] + e * w0
                    accs[j * C + 1] = accs[j * C + 1] + e * w1
                    accs[j * C + 2] = accs[j * C + 2] + e * w2
                return tuple(accs)

            init = (jnp.zeros((LANES,), jnp.float32),) * (RBLK * C)
            accs = lax.fori_loop(0, L, l_body, init)
            for j in range(RBLK):
                for c in range(C):
                    r = blk * RBLK + j
                    acc_v[pl.ds((r * C + c) * D, D)] = accs[j * C + c]

        b_vec = b_v[...]
        for t in range(GROUPS):
            a = []
            for c in range(C):
                base = lanes48 + (t * LANES * C * D + c * D)
                tot = jnp.full((LANES,), b_vec[c], jnp.float32)
                for d in range(D):
                    tot = tot + plsc.load_gather(acc_v, [base + d])
                a.append(tot)
            m = jnp.maximum(jnp.maximum(a[0], a[1]), a[2])
            s = (jnp.exp(a[0] - m) + jnp.exp(a[1] - m) + jnp.exp(a[2] - m))
            u = (s - 2.0) / (s + 2.0)
            u2 = u * u
            ln_s = LN2 + 2.0 * u * (1.0 + u2 * (1.0 / 3.0 + u2 * (
                1.0 / 5.0 + u2 * (1.0 / 7.0 + u2 * (1.0 / 9.0)))))
            ridx = lanes + t * LANES
            for c in range(C):
                plsc.store_scatter(out_v,
                                   [ridx, jnp.full((LANES,), c, jnp.int32)],
                                   a[c] - m - ln_s)
        pltpu.sync_copy(out_v, out_hbm.at[pl.ds(chunk * CHUNK_ROWS,
                                                CHUNK_ROWS)])

    issue(0, idx_v0, rows_v0, sem0)

    def pair_body(p, carry):
        g0 = 2 * p
        issue(g0 + 1, idx_v1, rows_v1, sem1)
        drain(idx_v0, rows_v0, sem0)
        compute(g0, rows_v0)

        @pl.when(p < CHUNKS_PER_TILE // 2 - 1)
        def _():
            issue(g0 + 2, idx_v0, rows_v0, sem0)

        drain(idx_v1, rows_v1, sem1)
        compute(g0 + 1, rows_v1)
        return carry

    lax.fori_loop(0, CHUNKS_PER_TILE // 2, pair_body, 0)


def kernel(x, emb, W, b):
    x_r = (x.astype(jnp.int32) * (ROW_STRIDE // D)).reshape(
        B_TOTAL * L // IDX_W, IDX_W)
    w_r = W.reshape(L, D, C).transpose(0, 2, 1).reshape(L * C * D)
    b_pad = jnp.zeros((LANES,), jnp.float32).at[:C].set(b)
    mesh = plsc.VectorSubcoreMesh(core_axis_name="c", subcore_axis_name="s")
    emb_rm = jnp.pad(emb, ((0, 0), (0, ROW_STRIDE - D))).reshape(
        EMB_ROWS * ROW_STRIDE // D, D)
    fn = pl.kernel(
        _body,
        mesh=mesh,
        compiler_params=pltpu.CompilerParams(
            needs_layout_passes=False, use_tc_tiling_on_sc=False),
        out_type=jax.ShapeDtypeStruct((B_TOTAL, C), jnp.float32),
        scratch_types=[
            pltpu.VMEM((NIDX, IDX_W), jnp.int32),
            pltpu.VMEM((NIDX, IDX_W), jnp.int32),
            pltpu.VMEM((LOOKUPS, D), jnp.float32),
            pltpu.VMEM((LOOKUPS, D), jnp.float32),
            pltpu.VMEM((L * C * D,), jnp.float32),
            pltpu.VMEM((LANES,), jnp.float32),
            pltpu.VMEM((CHUNK_ROWS * C * D,), jnp.float32),
            pltpu.VMEM((CHUNK_ROWS, C), jnp.float32),
            pltpu.SemaphoreType.DMA,
            pltpu.SemaphoreType.DMA,
        ],
    )
    return fn(x_r, emb_rm, w_r, b_pad)


# speedup vs baseline: 1.6305x; 1.6305x over previous
import jax
import jax.numpy as jnp
from jax import lax
from jax.experimental import pallas as pl
from jax.experimental.pallas import tpu as pltpu
from jax.experimental.pallas import tpu_sc as plsc

B_TOTAL = 16384
L = 50
D = 16
C = 3
LANES = 16
NUM_CORES = 2
NUM_SUBCORES = 16
NW = NUM_CORES * NUM_SUBCORES
ROWS_PER_TILE = B_TOTAL // NW
CHUNK_ROWS = 64
GROUPS = CHUNK_ROWS // LANES
CHUNKS_PER_TILE = ROWS_PER_TILE // CHUNK_ROWS
LOOKUPS = CHUNK_ROWS * L
IDX_W = 128
NIDX = LOOKUPS // IDX_W

LN2 = 0.6931471805599453

TCOLS = EMB_ROWS = 1000000
TBLK = 1000
NBLK_TOTAL = TCOLS // TBLK
MAX_BLK_PER_TILE = -(-NBLK_TOTAL // NW)
TUNROLL = 10


def _transpose_body(embt_hbm, out_hbm, in_v, out_v):
    cid = lax.axis_index("c")
    sid = lax.axis_index("s")
    wid = sid * NUM_CORES + cid
    lanes = lax.iota(jnp.int32, LANES)

    def blk_body(b, carry):
        blk = b * NW + wid

        @pl.when(blk < NBLK_TOTAL)
        def _():
            col0 = blk * TBLK
            pltpu.sync_copy(embt_hbm.at[:, pl.ds(col0, TBLK)], in_v)

            def col_body(i, carry):
                c = i * TUNROLL
                for k in range(TUNROLL):
                    vec = plsc.load_gather(
                        in_v, [lanes, jnp.full((LANES,), c + k, jnp.int32)])
                    out_v[pl.ds((c + k) * D, D)] = vec
                return carry

            lax.fori_loop(0, TBLK // TUNROLL, col_body, 0)
            pltpu.sync_copy(out_v, out_hbm.at[pl.ds(col0 * D, TBLK * D)])

        return carry

    lax.fori_loop(0, MAX_BLK_PER_TILE, blk_body, 0)


RBLK = 8
ROW_STRIDE = 128
DBLK = 2048


def _detile_body(x_ref, p_ref, o_ref):
    o_ref[...] = jax.lax.dot_general(
        x_ref[...], p_ref[...],
        dimension_numbers=(((0,), (0,)), ((), ())),
        preferred_element_type=jnp.float32)


def _body(x_hbm, emb_hbm, w_hbm, b_hbm, out_hbm,
          idx_v0, idx_v1, rows_v0, rows_v1, w_v, b_v, acc_v, out_v,
          sem0, sem1):
    cid = lax.axis_index("c")
    sid = lax.axis_index("s")
    wid = sid * NUM_CORES + cid

    pltpu.sync_copy(w_hbm, w_v)
    pltpu.sync_copy(b_hbm, b_v)
    lanes = lax.iota(jnp.int32, LANES)
    lanes48 = lanes * (C * D)

    def issue(g, idx_v, rows_v, sem):
        chunk = wid * CHUNKS_PER_TILE + g
        pltpu.sync_copy(x_hbm.at[pl.ds(chunk * NIDX, NIDX)], idx_v)
        for j in range(NIDX):
            pltpu.async_copy(emb_hbm.at[idx_v.at[j]],
                             rows_v.at[pl.ds(j * IDX_W, IDX_W)], sem)

    def drain(idx_v, rows_v, sem):
        for j in range(NIDX):
            pltpu.make_async_copy(emb_hbm.at[idx_v.at[j]],
                                  rows_v.at[pl.ds(j * IDX_W, IDX_W)],
                                  sem).wait()

    def compute(g, rows_v):
        chunk = wid * CHUNKS_PER_TILE + g
        for blk in range(CHUNK_ROWS // RBLK):
            def l_body(l, accs, blk=blk):
                accs = list(accs)
                w0 = w_v[pl.ds((l * C + 0) * D, D)]
                w1 = w_v[pl.ds((l * C + 1) * D, D)]
                w2 = w_v[pl.ds((l * C + 2) * D, D)]
                for j in range(RBLK):
                    r = blk * RBLK + j
                    e = rows_v[r * L + l]
                    accs[j * C + 0] = accs[j * C + 0] + e * w0
                    accs[j * C + 1] = accs[j * C + 1] + e * w1
                    accs[j * C + 2] = accs[j * C + 2] + e * w2
                return tuple(accs)

            init = (jnp.zeros((LANES,), jnp.float32),) * (RBLK * C)
            accs = lax.fori_loop(0, L, l_body, init)
            for j in range(RBLK):
                for c in range(C):
                    r = blk * RBLK + j
                    acc_v[pl.ds((r * C + c) * D, D)] = accs[j * C + c]

        b_vec = b_v[...]
        for t in range(GROUPS):
            a = []
            for c in range(C):
                base = lanes48 + (t * LANES * C * D + c * D)
                tot = jnp.full((LANES,), b_vec[c], jnp.float32)
                for d in range(D):
                    tot = tot + plsc.load_gather(acc_v, [base + d])
                a.append(tot)
            m = jnp.maximum(jnp.maximum(a[0], a[1]), a[2])
            s = (jnp.exp(a[0] - m) + jnp.exp(a[1] - m) + jnp.exp(a[2] - m))
            u = (s - 2.0) / (s + 2.0)
            u2 = u * u
            ln_s = LN2 + 2.0 * u * (1.0 + u2 * (1.0 / 3.0 + u2 * (
                1.0 / 5.0 + u2 * (1.0 / 7.0 + u2 * (1.0 / 9.0)))))
            ridx = lanes + t * LANES
            for c in range(C):
                plsc.store_scatter(out_v,
                                   [ridx, jnp.full((LANES,), c, jnp.int32)],
                                   a[c] - m - ln_s)
        pltpu.sync_copy(out_v, out_hbm.at[pl.ds(chunk * CHUNK_ROWS,
                                                CHUNK_ROWS)])

    issue(0, idx_v0, rows_v0, sem0)

    def pair_body(p, carry):
        g0 = 2 * p
        issue(g0 + 1, idx_v1, rows_v1, sem1)
        drain(idx_v0, rows_v0, sem0)
        compute(g0, rows_v0)

        @pl.when(p < CHUNKS_PER_TILE // 2 - 1)
        def _():
            issue(g0 + 2, idx_v0, rows_v0, sem0)

        drain(idx_v1, rows_v1, sem1)
        compute(g0 + 1, rows_v1)
        return carry

    lax.fori_loop(0, CHUNKS_PER_TILE // 2, pair_body, 0)


def kernel(x, emb, W, b):
    x_r = (x.astype(jnp.int32) * (ROW_STRIDE // D)).reshape(
        B_TOTAL * L // IDX_W, IDX_W)
    w_r = W.reshape(L, D, C).transpose(0, 2, 1).reshape(L * C * D)
    b_pad = jnp.zeros((LANES,), jnp.float32).at[:C].set(b)
    mesh = plsc.VectorSubcoreMesh(core_axis_name="c", subcore_axis_name="s")
    sel = jnp.eye(D, ROW_STRIDE, dtype=jnp.float32)
    detile = pl.pallas_call(
        _detile_body,
        grid=(-(-TCOLS // DBLK),),
        in_specs=[pl.BlockSpec((D, DBLK), lambda i: (0, i)),
                  pl.BlockSpec((D, ROW_STRIDE), lambda i: (0, 0))],
        out_specs=pl.BlockSpec((DBLK, ROW_STRIDE), lambda i: (i, 0)),
        out_shape=jax.ShapeDtypeStruct((EMB_ROWS, ROW_STRIDE), jnp.float32),
    )
    emb_rm = detile(emb.T, sel).reshape(EMB_ROWS * ROW_STRIDE // D, D)
    fn = pl.kernel(
        _body,
        mesh=mesh,
        compiler_params=pltpu.CompilerParams(
            needs_layout_passes=False, use_tc_tiling_on_sc=False),
        out_type=jax.ShapeDtypeStruct((B_TOTAL, C), jnp.float32),
        scratch_types=[
            pltpu.VMEM((NIDX, IDX_W), jnp.int32),
            pltpu.VMEM((NIDX, IDX_W), jnp.int32),
            pltpu.VMEM((LOOKUPS, D), jnp.float32),
            pltpu.VMEM((LOOKUPS, D), jnp.float32),
            pltpu.VMEM((L * C * D,), jnp.float32),
            pltpu.VMEM((LANES,), jnp.float32),
            pltpu.VMEM((CHUNK_ROWS * C * D,), jnp.float32),
            pltpu.VMEM((CHUNK_ROWS, C), jnp.float32),
            pltpu.SemaphoreType.DMA,
            pltpu.SemaphoreType.DMA,
        ],
    )
    return fn(x_r, emb_rm, w_r, b_pad)


# speedup vs baseline: 2.6574x; 1.6298x over previous
import jax
import jax.numpy as jnp
from jax import lax
from jax.experimental import pallas as pl
from jax.experimental.pallas import tpu as pltpu
from jax.experimental.pallas import tpu_sc as plsc

B_TOTAL = 16384
L = 50
D = 16
C = 3
LANES = 16
NUM_CORES = 2
NUM_SUBCORES = 16
NW = NUM_CORES * NUM_SUBCORES
ROWS_PER_TILE = B_TOTAL // NW
CHUNK_ROWS = 64
GROUPS = CHUNK_ROWS // LANES
CHUNKS_PER_TILE = ROWS_PER_TILE // CHUNK_ROWS
LOOKUPS = CHUNK_ROWS * L
IDX_W = 128
NIDX = LOOKUPS // IDX_W

LN2 = 0.6931471805599453

TCOLS = EMB_ROWS = 1000000
TBLK = 1000
NBLK_TOTAL = TCOLS // TBLK
MAX_BLK_PER_TILE = -(-NBLK_TOTAL // NW)
TUNROLL = 10


def _transpose_body(embt_hbm, out_hbm, in_v, out_v):
    cid = lax.axis_index("c")
    sid = lax.axis_index("s")
    wid = sid * NUM_CORES + cid
    lanes = lax.iota(jnp.int32, LANES)

    def blk_body(b, carry):
        blk = b * NW + wid

        @pl.when(blk < NBLK_TOTAL)
        def _():
            col0 = blk * TBLK
            pltpu.sync_copy(embt_hbm.at[:, pl.ds(col0, TBLK)], in_v)

            def col_body(i, carry):
                c = i * TUNROLL
                for k in range(TUNROLL):
                    vec = plsc.load_gather(
                        in_v, [lanes, jnp.full((LANES,), c + k, jnp.int32)])
                    out_v[pl.ds((c + k) * D, D)] = vec
                return carry

            lax.fori_loop(0, TBLK // TUNROLL, col_body, 0)
            pltpu.sync_copy(out_v, out_hbm.at[pl.ds(col0 * D, TBLK * D)])

        return carry

    lax.fori_loop(0, MAX_BLK_PER_TILE, blk_body, 0)


RBLK = 8
ROW_STRIDE = 128
DBLK = 8192


def _detile_body(x_ref, p_ref, o_ref):
    o_ref[...] = jax.lax.dot_general(
        x_ref[...], p_ref[...],
        dimension_numbers=(((0,), (0,)), ((), ())),
        preferred_element_type=jnp.float32)


def _body(x_hbm, emb_hbm, w_hbm, b_hbm, out_hbm,
          idx_v0, idx_v1, rows_v0, rows_v1, w_v, b_v, acc_v, out_v,
          sem0, sem1):
    cid = lax.axis_index("c")
    sid = lax.axis_index("s")
    wid = sid * NUM_CORES + cid

    pltpu.sync_copy(w_hbm, w_v)
    pltpu.sync_copy(b_hbm, b_v)
    lanes = lax.iota(jnp.int32, LANES)
    lanes48 = lanes * (C * D)

    def issue(g, idx_v, rows_v, sem):
        chunk = wid * CHUNKS_PER_TILE + g
        pltpu.sync_copy(x_hbm.at[pl.ds(chunk * NIDX, NIDX)], idx_v)
        for j in range(NIDX):
            pltpu.async_copy(emb_hbm.at[idx_v.at[j]],
                             rows_v.at[pl.ds(j * IDX_W, IDX_W)], sem)

    def drain(idx_v, rows_v, sem):
        for j in range(NIDX):
            pltpu.make_async_copy(emb_hbm.at[idx_v.at[j]],
                                  rows_v.at[pl.ds(j * IDX_W, IDX_W)],
                                  sem).wait()

    def compute(g, rows_v):
        chunk = wid * CHUNKS_PER_TILE + g
        for blk in range(CHUNK_ROWS // RBLK):
            def l_body(l, accs, blk=blk):
                accs = list(accs)
                w0 = w_v[pl.ds((l * C + 0) * D, D)]
                w1 = w_v[pl.ds((l * C + 1) * D, D)]
                w2 = w_v[pl.ds((l * C + 2) * D, D)]
                for j in range(RBLK):
                    r = blk * RBLK + j
                    e = rows_v[r * L + l]
                    accs[j * C + 0] = accs[j * C + 0] + e * w0
                    accs[j * C + 1] = accs[j * C + 1] + e * w1
                    accs[j * C + 2] = accs[j * C + 2] + e * w2
                return tuple(accs)

            init = (jnp.zeros((LANES,), jnp.float32),) * (RBLK * C)
            accs = lax.fori_loop(0, L, l_body, init)
            for j in range(RBLK):
                for c in range(C):
                    r = blk * RBLK + j
                    acc_v[pl.ds((r * C + c) * D, D)] = accs[j * C + c]

        b_vec = b_v[...]
        for t in range(GROUPS):
            a = []
            for c in range(C):
                base = lanes48 + (t * LANES * C * D + c * D)
                tot = jnp.full((LANES,), b_vec[c], jnp.float32)
                for d in range(D):
                    tot = tot + plsc.load_gather(acc_v, [base + d])
                a.append(tot)
            m = jnp.maximum(jnp.maximum(a[0], a[1]), a[2])
            s = (jnp.exp(a[0] - m) + jnp.exp(a[1] - m) + jnp.exp(a[2] - m))
            u = (s - 2.0) / (s + 2.0)
            u2 = u * u
            ln_s = LN2 + 2.0 * u * (1.0 + u2 * (1.0 / 3.0 + u2 * (
                1.0 / 5.0 + u2 * (1.0 / 7.0 + u2 * (1.0 / 9.0)))))
            ridx = lanes + t * LANES
            for c in range(C):
                plsc.store_scatter(out_v,
                                   [ridx, jnp.full((LANES,), c, jnp.int32)],
                                   a[c] - m - ln_s)
        pltpu.sync_copy(out_v, out_hbm.at[pl.ds(chunk * CHUNK_ROWS,
                                                CHUNK_ROWS)])

    issue(0, idx_v0, rows_v0, sem0)

    def pair_body(p, carry):
        g0 = 2 * p
        issue(g0 + 1, idx_v1, rows_v1, sem1)
        drain(idx_v0, rows_v0, sem0)
        compute(g0, rows_v0)

        @pl.when(p < CHUNKS_PER_TILE // 2 - 1)
        def _():
            issue(g0 + 2, idx_v0, rows_v0, sem0)

        drain(idx_v1, rows_v1, sem1)
        compute(g0 + 1, rows_v1)
        return carry

    lax.fori_loop(0, CHUNKS_PER_TILE // 2, pair_body, 0)


def kernel(x, emb, W, b):
    x_r = (x.astype(jnp.int32) * (ROW_STRIDE // D)).reshape(
        B_TOTAL * L // IDX_W, IDX_W)
    w_r = W.reshape(L, D, C).transpose(0, 2, 1).reshape(L * C * D)
    b_pad = jnp.zeros((LANES,), jnp.float32).at[:C].set(b)
    mesh = plsc.VectorSubcoreMesh(core_axis_name="c", subcore_axis_name="s")
    sel = jnp.eye(D, ROW_STRIDE, dtype=jnp.float32)
    detile = pl.pallas_call(
        _detile_body,
        grid=(-(-TCOLS // DBLK),),
        in_specs=[pl.BlockSpec((D, DBLK), lambda i: (0, i)),
                  pl.BlockSpec((D, ROW_STRIDE), lambda i: (0, 0))],
        out_specs=pl.BlockSpec((DBLK, ROW_STRIDE), lambda i: (i, 0)),
        out_shape=jax.ShapeDtypeStruct((EMB_ROWS, ROW_STRIDE), jnp.float32),
    )
    emb_rm = detile(emb.T, sel).reshape(EMB_ROWS * ROW_STRIDE // D, D)
    fn = pl.kernel(
        _body,
        mesh=mesh,
        compiler_params=pltpu.CompilerParams(
            needs_layout_passes=False, use_tc_tiling_on_sc=False),
        out_type=jax.ShapeDtypeStruct((B_TOTAL, C), jnp.float32),
        scratch_types=[
            pltpu.VMEM((NIDX, IDX_W), jnp.int32),
            pltpu.VMEM((NIDX, IDX_W), jnp.int32),
            pltpu.VMEM((LOOKUPS, D), jnp.float32),
            pltpu.VMEM((LOOKUPS, D), jnp.float32),
            pltpu.VMEM((L * C * D,), jnp.float32),
            pltpu.VMEM((LANES,), jnp.float32),
            pltpu.VMEM((CHUNK_ROWS * C * D,), jnp.float32),
            pltpu.VMEM((CHUNK_ROWS, C), jnp.float32),
            pltpu.SemaphoreType.DMA,
            pltpu.SemaphoreType.DMA,
        ],
    )
    return fn(x_r, emb_rm, w_r, b_pad)


# speedup vs baseline: 2.9915x; 1.1257x over previous
import jax
import jax.numpy as jnp
from jax import lax
from jax.experimental import pallas as pl
from jax.experimental.pallas import tpu as pltpu
from jax.experimental.pallas import tpu_sc as plsc

B_TOTAL = 16384
L = 50
D = 16
C = 3
LANES = 16
NUM_CORES = 2
NUM_SUBCORES = 16
NW = NUM_CORES * NUM_SUBCORES
ROWS_PER_TILE = B_TOTAL // NW
CHUNK_ROWS = 64
GROUPS = CHUNK_ROWS // LANES
CHUNKS_PER_TILE = ROWS_PER_TILE // CHUNK_ROWS
LOOKUPS = CHUNK_ROWS * L
IDX_W = 128
NIDX = LOOKUPS // IDX_W

LN2 = 0.6931471805599453

TCOLS = EMB_ROWS = 1000000
TBLK = 1000
NBLK_TOTAL = TCOLS // TBLK
MAX_BLK_PER_TILE = -(-NBLK_TOTAL // NW)
TUNROLL = 10


def _transpose_body(embt_hbm, out_hbm, in_v, out_v):
    cid = lax.axis_index("c")
    sid = lax.axis_index("s")
    wid = sid * NUM_CORES + cid
    lanes = lax.iota(jnp.int32, LANES)

    def blk_body(b, carry):
        blk = b * NW + wid

        @pl.when(blk < NBLK_TOTAL)
        def _():
            col0 = blk * TBLK
            pltpu.sync_copy(embt_hbm.at[:, pl.ds(col0, TBLK)], in_v)

            def col_body(i, carry):
                c = i * TUNROLL
                for k in range(TUNROLL):
                    vec = plsc.load_gather(
                        in_v, [lanes, jnp.full((LANES,), c + k, jnp.int32)])
                    out_v[pl.ds((c + k) * D, D)] = vec
                return carry

            lax.fori_loop(0, TBLK // TUNROLL, col_body, 0)
            pltpu.sync_copy(out_v, out_hbm.at[pl.ds(col0 * D, TBLK * D)])

        return carry

    lax.fori_loop(0, MAX_BLK_PER_TILE, blk_body, 0)


RBLK = 8
ROW_STRIDE = 128
DBLK = 16384


def _detile_body(x_ref, p_ref, o_ref):
    o_ref[...] = jax.lax.dot_general(
        x_ref[...], p_ref[...],
        dimension_numbers=(((0,), (0,)), ((), ())),
        preferred_element_type=jnp.float32)


def _body(x_hbm, emb_hbm, w_hbm, b_hbm, out_hbm,
          idx_v0, idx_v1, rows_v0, rows_v1, w_v, b_v, acc_v, out_v,
          sem0, sem1):
    cid = lax.axis_index("c")
    sid = lax.axis_index("s")
    wid = sid * NUM_CORES + cid

    pltpu.sync_copy(w_hbm, w_v)
    pltpu.sync_copy(b_hbm, b_v)
    lanes = lax.iota(jnp.int32, LANES)
    lanes48 = lanes * (C * D)

    def issue(g, idx_v, rows_v, sem):
        chunk = wid * CHUNKS_PER_TILE + g
        pltpu.sync_copy(x_hbm.at[pl.ds(chunk * NIDX, NIDX)], idx_v)
        for j in range(NIDX):
            pltpu.async_copy(emb_hbm.at[idx_v.at[j]],
                             rows_v.at[pl.ds(j * IDX_W, IDX_W)], sem)

    def drain(idx_v, rows_v, sem):
        for j in range(NIDX):
            pltpu.make_async_copy(emb_hbm.at[idx_v.at[j]],
                                  rows_v.at[pl.ds(j * IDX_W, IDX_W)],
                                  sem).wait()

    def compute(g, rows_v):
        chunk = wid * CHUNKS_PER_TILE + g
        for blk in range(CHUNK_ROWS // RBLK):
            def l_body(l, accs, blk=blk):
                accs = list(accs)
                w0 = w_v[pl.ds((l * C + 0) * D, D)]
                w1 = w_v[pl.ds((l * C + 1) * D, D)]
                w2 = w_v[pl.ds((l * C + 2) * D, D)]
                for j in range(RBLK):
                    r = blk * RBLK + j
                    e = rows_v[r * L + l]
                    accs[j * C + 0] = accs[j * C + 0] + e * w0
                    accs[j * C + 1] = accs[j * C + 1] + e * w1
                    accs[j * C + 2] = accs[j * C + 2] + e * w2
                return tuple(accs)

            init = (jnp.zeros((LANES,), jnp.float32),) * (RBLK * C)
            accs = lax.fori_loop(0, L, l_body, init)
            for j in range(RBLK):
                for c in range(C):
                    r = blk * RBLK + j
                    acc_v[pl.ds((r * C + c) * D, D)] = accs[j * C + c]

        b_vec = b_v[...]
        for t in range(GROUPS):
            a = []
            for c in range(C):
                base = lanes48 + (t * LANES * C * D + c * D)
                tot = jnp.full((LANES,), b_vec[c], jnp.float32)
                for d in range(D):
                    tot = tot + plsc.load_gather(acc_v, [base + d])
                a.append(tot)
            m = jnp.maximum(jnp.maximum(a[0], a[1]), a[2])
            s = (jnp.exp(a[0] - m) + jnp.exp(a[1] - m) + jnp.exp(a[2] - m))
            u = (s - 2.0) / (s + 2.0)
            u2 = u * u
            ln_s = LN2 + 2.0 * u * (1.0 + u2 * (1.0 / 3.0 + u2 * (
                1.0 / 5.0 + u2 * (1.0 / 7.0 + u2 * (1.0 / 9.0)))))
            ridx = lanes + t * LANES
            for c in range(C):
                plsc.store_scatter(out_v,
                                   [ridx, jnp.full((LANES,), c, jnp.int32)],
                                   a[c] - m - ln_s)
        pltpu.sync_copy(out_v, out_hbm.at[pl.ds(chunk * CHUNK_ROWS,
                                                CHUNK_ROWS)])

    issue(0, idx_v0, rows_v0, sem0)

    def pair_body(p, carry):
        g0 = 2 * p
        issue(g0 + 1, idx_v1, rows_v1, sem1)
        drain(idx_v0, rows_v0, sem0)
        compute(g0, rows_v0)

        @pl.when(p < CHUNKS_PER_TILE // 2 - 1)
        def _():
            issue(g0 + 2, idx_v0, rows_v0, sem0)

        drain(idx_v1, rows_v1, sem1)
        compute(g0 + 1, rows_v1)
        return carry

    lax.fori_loop(0, CHUNKS_PER_TILE // 2, pair_body, 0)


def kernel(x, emb, W, b):
    x_r = (x.astype(jnp.int32) * (ROW_STRIDE // D)).reshape(
        B_TOTAL * L // IDX_W, IDX_W)
    w_r = W.reshape(L, D, C).transpose(0, 2, 1).reshape(L * C * D)
    b_pad = jnp.zeros((LANES,), jnp.float32).at[:C].set(b)
    mesh = plsc.VectorSubcoreMesh(core_axis_name="c", subcore_axis_name="s")
    sel = jnp.eye(D, ROW_STRIDE, dtype=jnp.float32)
    detile = pl.pallas_call(
        _detile_body,
        grid=(-(-TCOLS // DBLK),),
        in_specs=[pl.BlockSpec((D, DBLK), lambda i: (0, i)),
                  pl.BlockSpec((D, ROW_STRIDE), lambda i: (0, 0))],
        out_specs=pl.BlockSpec((DBLK, ROW_STRIDE), lambda i: (i, 0)),
        out_shape=jax.ShapeDtypeStruct((EMB_ROWS, ROW_STRIDE), jnp.float32),
    )
    emb_rm = detile(emb.T, sel).reshape(EMB_ROWS * ROW_STRIDE // D, D)
    fn = pl.kernel(
        _body,
        mesh=mesh,
        compiler_params=pltpu.CompilerParams(
            needs_layout_passes=False, use_tc_tiling_on_sc=False),
        out_type=jax.ShapeDtypeStruct((B_TOTAL, C), jnp.float32),
        scratch_types=[
            pltpu.VMEM((NIDX, IDX_W), jnp.int32),
            pltpu.VMEM((NIDX, IDX_W), jnp.int32),
            pltpu.VMEM((LOOKUPS, D), jnp.float32),
            pltpu.VMEM((LOOKUPS, D), jnp.float32),
            pltpu.VMEM((L * C * D,), jnp.float32),
            pltpu.VMEM((LANES,), jnp.float32),
            pltpu.VMEM((CHUNK_ROWS * C * D,), jnp.float32),
            pltpu.VMEM((CHUNK_ROWS, C), jnp.float32),
            pltpu.SemaphoreType.DMA,
            pltpu.SemaphoreType.DMA,
        ],
    )
    return fn(x_r, emb_rm, w_r, b_pad)


# speedup vs baseline: 3.0614x; 1.0234x over previous
import jax
import jax.numpy as jnp
from jax import lax
from jax.experimental import pallas as pl
from jax.experimental.pallas import tpu as pltpu
from jax.experimental.pallas import tpu_sc as plsc

B_TOTAL = 16384
L = 50
D = 16
C = 3
LANES = 16
NUM_CORES = 2
NUM_SUBCORES = 16
NW = NUM_CORES * NUM_SUBCORES
ROWS_PER_TILE = B_TOTAL // NW
CHUNK_ROWS = 64
GROUPS = CHUNK_ROWS // LANES
CHUNKS_PER_TILE = ROWS_PER_TILE // CHUNK_ROWS
LOOKUPS = CHUNK_ROWS * L
IDX_W = 128
NIDX = LOOKUPS // IDX_W

LN2 = 0.6931471805599453

TCOLS = EMB_ROWS = 1000000
TBLK = 1000
NBLK_TOTAL = TCOLS // TBLK
MAX_BLK_PER_TILE = -(-NBLK_TOTAL // NW)
TUNROLL = 10


def _transpose_body(embt_hbm, out_hbm, in_v, out_v):
    cid = lax.axis_index("c")
    sid = lax.axis_index("s")
    wid = sid * NUM_CORES + cid
    lanes = lax.iota(jnp.int32, LANES)

    def blk_body(b, carry):
        blk = b * NW + wid

        @pl.when(blk < NBLK_TOTAL)
        def _():
            col0 = blk * TBLK
            pltpu.sync_copy(embt_hbm.at[:, pl.ds(col0, TBLK)], in_v)

            def col_body(i, carry):
                c = i * TUNROLL
                for k in range(TUNROLL):
                    vec = plsc.load_gather(
                        in_v, [lanes, jnp.full((LANES,), c + k, jnp.int32)])
                    out_v[pl.ds((c + k) * D, D)] = vec
                return carry

            lax.fori_loop(0, TBLK // TUNROLL, col_body, 0)
            pltpu.sync_copy(out_v, out_hbm.at[pl.ds(col0 * D, TBLK * D)])

        return carry

    lax.fori_loop(0, MAX_BLK_PER_TILE, blk_body, 0)


RBLK = 8
ROW_STRIDE = 128
DBLK = 32768


def _detile_body(x_ref, p_ref, o_ref):
    o_ref[...] = jax.lax.dot_general(
        x_ref[...], p_ref[...],
        dimension_numbers=(((0,), (0,)), ((), ())),
        preferred_element_type=jnp.float32)


def _body(x_hbm, emb_hbm, w_hbm, b_hbm, out_hbm,
          idx_v0, idx_v1, rows_v0, rows_v1, w_v, b_v, acc_v, out_v,
          sem0, sem1):
    cid = lax.axis_index("c")
    sid = lax.axis_index("s")
    wid = sid * NUM_CORES + cid

    pltpu.sync_copy(w_hbm, w_v)
    pltpu.sync_copy(b_hbm, b_v)
    lanes = lax.iota(jnp.int32, LANES)
    lanes48 = lanes * (C * D)

    def issue(g, idx_v, rows_v, sem):
        chunk = wid * CHUNKS_PER_TILE + g
        pltpu.sync_copy(x_hbm.at[pl.ds(chunk * NIDX, NIDX)], idx_v)
        for j in range(NIDX):
            pltpu.async_copy(emb_hbm.at[idx_v.at[j]],
                             rows_v.at[pl.ds(j * IDX_W, IDX_W)], sem)

    def drain(idx_v, rows_v, sem):
        for j in range(NIDX):
            pltpu.make_async_copy(emb_hbm.at[idx_v.at[j]],
                                  rows_v.at[pl.ds(j * IDX_W, IDX_W)],
                                  sem).wait()

    def compute(g, rows_v):
        chunk = wid * CHUNKS_PER_TILE + g
        for blk in range(CHUNK_ROWS // RBLK):
            def l_body(l, accs, blk=blk):
                accs = list(accs)
                w0 = w_v[pl.ds((l * C + 0) * D, D)]
                w1 = w_v[pl.ds((l * C + 1) * D, D)]
                w2 = w_v[pl.ds((l * C + 2) * D, D)]
                for j in range(RBLK):
                    r = blk * RBLK + j
                    e = rows_v[r * L + l]
                    accs[j * C + 0] = accs[j * C + 0] + e * w0
                    accs[j * C + 1] = accs[j * C + 1] + e * w1
                    accs[j * C + 2] = accs[j * C + 2] + e * w2
                return tuple(accs)

            init = (jnp.zeros((LANES,), jnp.float32),) * (RBLK * C)
            accs = lax.fori_loop(0, L, l_body, init)
            for j in range(RBLK):
                for c in range(C):
                    r = blk * RBLK + j
                    acc_v[pl.ds((r * C + c) * D, D)] = accs[j * C + c]

        b_vec = b_v[...]
        for t in range(GROUPS):
            a = []
            for c in range(C):
                base = lanes48 + (t * LANES * C * D + c * D)
                tot = jnp.full((LANES,), b_vec[c], jnp.float32)
                for d in range(D):
                    tot = tot + plsc.load_gather(acc_v, [base + d])
                a.append(tot)
            m = jnp.maximum(jnp.maximum(a[0], a[1]), a[2])
            s = (jnp.exp(a[0] - m) + jnp.exp(a[1] - m) + jnp.exp(a[2] - m))
            u = (s - 2.0) / (s + 2.0)
            u2 = u * u
            ln_s = LN2 + 2.0 * u * (1.0 + u2 * (1.0 / 3.0 + u2 * (
                1.0 / 5.0 + u2 * (1.0 / 7.0 + u2 * (1.0 / 9.0)))))
            ridx = lanes + t * LANES
            for c in range(C):
                plsc.store_scatter(out_v,
                                   [ridx, jnp.full((LANES,), c, jnp.int32)],
                                   a[c] - m - ln_s)
        pltpu.sync_copy(out_v, out_hbm.at[pl.ds(chunk * CHUNK_ROWS,
                                                CHUNK_ROWS)])

    issue(0, idx_v0, rows_v0, sem0)

    def pair_body(p, carry):
        g0 = 2 * p
        issue(g0 + 1, idx_v1, rows_v1, sem1)
        drain(idx_v0, rows_v0, sem0)
        compute(g0, rows_v0)

        @pl.when(p < CHUNKS_PER_TILE // 2 - 1)
        def _():
            issue(g0 + 2, idx_v0, rows_v0, sem0)

        drain(idx_v1, rows_v1, sem1)
        compute(g0 + 1, rows_v1)
        return carry

    lax.fori_loop(0, CHUNKS_PER_TILE // 2, pair_body, 0)


def kernel(x, emb, W, b):
    x_r = (x.astype(jnp.int32) * (ROW_STRIDE // D)).reshape(
        B_TOTAL * L // IDX_W, IDX_W)
    w_r = W.reshape(L, D, C).transpose(0, 2, 1).reshape(L * C * D)
    b_pad = jnp.zeros((LANES,), jnp.float32).at[:C].set(b)
    mesh = plsc.VectorSubcoreMesh(core_axis_name="c", subcore_axis_name="s")
    sel = jnp.eye(D, ROW_STRIDE, dtype=jnp.float32)
    detile = pl.pallas_call(
        _detile_body,
        grid=(-(-TCOLS // DBLK),),
        in_specs=[pl.BlockSpec((D, DBLK), lambda i: (0, i)),
                  pl.BlockSpec((D, ROW_STRIDE), lambda i: (0, 0))],
        out_specs=pl.BlockSpec((DBLK, ROW_STRIDE), lambda i: (i, 0)),
        out_shape=jax.ShapeDtypeStruct((EMB_ROWS, ROW_STRIDE), jnp.float32),
    )
    emb_rm = detile(emb.T, sel).reshape(EMB_ROWS * ROW_STRIDE // D, D)
    fn = pl.kernel(
        _body,
        mesh=mesh,
        compiler_params=pltpu.CompilerParams(
            needs_layout_passes=False, use_tc_tiling_on_sc=False),
        out_type=jax.ShapeDtypeStruct((B_TOTAL, C), jnp.float32),
        scratch_types=[
            pltpu.VMEM((NIDX, IDX_W), jnp.int32),
            pltpu.VMEM((NIDX, IDX_W), jnp.int32),
            pltpu.VMEM((LOOKUPS, D), jnp.float32),
            pltpu.VMEM((LOOKUPS, D), jnp.float32),
            pltpu.VMEM((L * C * D,), jnp.float32),
            pltpu.VMEM((LANES,), jnp.float32),
            pltpu.VMEM((CHUNK_ROWS * C * D,), jnp.float32),
            pltpu.VMEM((CHUNK_ROWS, C), jnp.float32),
            pltpu.SemaphoreType.DMA,
            pltpu.SemaphoreType.DMA,
        ],
    )
    return fn(x_r, emb_rm, w_r, b_pad)
